# Initial kernel scaffold; baseline (speedup 1.0000x reference)
import jax
import jax.numpy as jnp
from jax.experimental import pallas as pl


def kernel(x, W, gamma, beta):
    raise NotImplementedError("write your pallas kernel here")


# speedup vs baseline: 12.1006x; 12.1006x over previous
import functools

import jax
import jax.numpy as jnp
from jax import lax
from jax.experimental import pallas as pl
from jax.experimental.pallas import tpu as pltpu
from jax.experimental.pallas import tpu_sc as plsc

KNN = 20
KPAD = 32


def _knn_body(n_pts, rows, xa_ref, xr_ref, wd_ref, w2_ref,
              idx_ref, a_ref, bb_ref):
    b = pl.program_id(0)
    xa = xa_ref[0]
    xr = xr_ref[0]
    g = jnp.dot(xr, xa.T, preferred_element_type=jnp.float32)
    xx_a = jnp.sum(xa * xa, axis=1)
    xx_r = jnp.sum(xr * xr, axis=1)
    d = 2.0 * g - xx_r[:, None] - xx_a[None, :]
    cols = lax.broadcasted_iota(jnp.int32, (rows, n_pts), 1)
    base = b * n_pts
    for t in range(KNN):
        mx = jnp.max(d, axis=1, keepdims=True)
        amin = jnp.min(jnp.where(d == mx, cols, n_pts), axis=1)
        idx_ref[0, t, :] = amin + base
        d = jnp.where(cols == amin[:, None], -jnp.inf, d)
    a_ref[0] = jnp.dot(xr, wd_ref[...].T, preferred_element_type=jnp.float32)
    bb = jnp.dot(xr, w2_ref[...].T, preferred_element_type=jnp.float32)
    bb_ref[0] = jnp.concatenate([bb, jnp.zeros_like(bb)], axis=1)


def _run_knn(xt, wd, w2, rows=256):
    b, n, c = xt.shape
    cout = wd.shape[0]
    grid = (b, n // rows)
    return pl.pallas_call(
        functools.partial(_knn_body, n, rows),
        grid=grid,
        in_specs=[
            pl.BlockSpec((1, n, c), lambda bi, i: (bi, 0, 0)),
            pl.BlockSpec((1, rows, c), lambda bi, i: (bi, i, 0)),
            pl.BlockSpec((cout, c), lambda bi, i: (0, 0)),
            pl.BlockSpec((cout, c), lambda bi, i: (0, 0)),
        ],
        out_specs=[
            pl.BlockSpec((1, KPAD, rows), lambda bi, i: (bi, 0, i)),
            pl.BlockSpec((1, rows, cout), lambda bi, i: (bi, i, 0)),
            pl.BlockSpec((1, rows, 2 * cout), lambda bi, i: (bi, i, 0)),
        ],
        out_shape=[
            jax.ShapeDtypeStruct((b, KPAD, n), jnp.int32),
            jax.ShapeDtypeStruct((b, n, cout), jnp.float32),
            jax.ShapeDtypeStruct((b, n, 2 * cout), jnp.float32),
        ],
    )(xt, xt, wd, w2)


_P = 32
_GROUPS = 5
_LANE = 16


def _make_sc_gather_reduce(npts, cout, nworkers):
    per_w = npts // nworkers
    nsub = per_w // _P
    idx_rows_per_sub = (_P * KNN) // 128
    idx_rows_per_w = (per_w * KNN) // 128
    mesh = plsc.VectorSubcoreMesh(core_axis_name="c", subcore_axis_name="s")
    nc = plsc.get_sparse_core_info().num_cores
    ngrp = cout // _LANE
    tw = 2 * cout

    @functools.partial(
        pl.kernel, mesh=mesh,
        out_type=[
            jax.ShapeDtypeStruct((npts, tw), jnp.float32),
            jax.ShapeDtypeStruct((npts, tw), jnp.float32),
            jax.ShapeDtypeStruct((npts, tw), jnp.float32),
            jax.ShapeDtypeStruct((nworkers * 8, tw), jnp.float32),
        ],
        scratch_types=[
            pltpu.VMEM((idx_rows_per_w, 128), jnp.int32),
            pltpu.VMEM((_GROUPS, 128, tw), jnp.float32),
            pltpu.VMEM((_P, tw), jnp.float32),
            pltpu.VMEM((_P, tw), jnp.float32),
            pltpu.VMEM((_P, tw), jnp.float32),
            pltpu.VMEM((8, tw), jnp.float32),
            pltpu.SemaphoreType.DMA,
        ],
    )
    def sc_kernel(bb_hbm, idx_hbm, s_hbm, mx_hbm, mn_hbm, s2_hbm,
                  idx_v, rows_v, sv, mv, nv, s2v, sem):
        wid = lax.axis_index("s") * nc + lax.axis_index("c")
        pltpu.sync_copy(idx_hbm.at[pl.ds(wid * idx_rows_per_w,
                                         idx_rows_per_w)], idx_v)

        def sub_body(sub, s2c):
            base_pt = wid * per_w + sub * _P
            handles = [
                pltpu.async_copy(
                    bb_hbm.at[idx_v.at[sub * idx_rows_per_sub + g]],
                    rows_v.at[g], sem)
                for g in range(_GROUPS)
            ]
            for h in handles:
                h.wait()

            def p_body(p, s2i):
                s2i = list(s2i)
                s = [None] * ngrp
                mxa = [None] * ngrp
                mna = [None] * ngrp
                e0 = p * KNN
                for t in range(KNN):
                    e = e0 + t
                    g = e // 128
                    r = e - g * 128
                    for cgi in range(ngrp):
                        v = rows_v[g, r, pl.ds(cgi * _LANE, _LANE)]
                        if t == 0:
                            s[cgi] = v
                            mxa[cgi] = v
                            mna[cgi] = v
                        else:
                            s[cgi] = s[cgi] + v
                            mxa[cgi] = jnp.maximum(mxa[cgi], v)
                            mna[cgi] = jnp.minimum(mna[cgi], v)
                        s2i[cgi] = s2i[cgi] + v * v
                for cgi in range(ngrp):
                    sv[p, pl.ds(cgi * _LANE, _LANE)] = s[cgi]
                    mv[p, pl.ds(cgi * _LANE, _LANE)] = mxa[cgi]
                    nv[p, pl.ds(cgi * _LANE, _LANE)] = mna[cgi]
                return tuple(s2i)

            s2c = lax.fori_loop(0, _P, p_body, s2c)
            pltpu.sync_copy(sv, s_hbm.at[pl.ds(base_pt, _P)])
            pltpu.sync_copy(mv, mx_hbm.at[pl.ds(base_pt, _P)])
            pltpu.sync_copy(nv, mn_hbm.at[pl.ds(base_pt, _P)])
            return s2c

        zero = jnp.zeros((_LANE,), jnp.float32)
        s2c = lax.fori_loop(0, nsub, sub_body, (zero,) * ngrp)
        for r in range(8):
            for cgi in range(ngrp):
                s2v[r, pl.ds(cgi * _LANE, _LANE)] = (
                    s2c[cgi] if r == 0 else zero)
        pltpu.sync_copy(s2v, s2_hbm.at[pl.ds(wid * 8, 8)])

    return sc_kernel


def _finalize_body(count, cout, a_ref, s_ref, mx_ref, mn_ref, s2_ref,
                   gam_ref, bet_ref, out_ref, acc_ref):
    phase = pl.program_id(0)
    i = pl.program_id(1)

    @pl.when(jnp.logical_and(phase == 0, i == 0))
    def _():
        acc_ref[...] = jnp.zeros_like(acc_ref)

    @pl.when(phase == 0)
    def _():
        a = a_ref[...]
        s = s_ref[:, :cout]
        acc_ref[0, :] += jnp.sum(a, axis=0)
        acc_ref[1, :] += jnp.sum(a * a, axis=0)
        acc_ref[2, :] += jnp.sum(a * s, axis=0)
        acc_ref[3, :] += jnp.sum(s, axis=0)

    @pl.when(phase == 1)
    def _():
        gam = gam_ref[0]
        bet = bet_ref[0]
        s2sum = jnp.sum(s2_ref[:, :cout], axis=0)
        sumh = KNN * acc_ref[0, :] + acc_ref[3, :]
        sumh2 = KNN * acc_ref[1, :] + 2.0 * acc_ref[2, :] + s2sum
        mean = sumh / count
        var = sumh2 / count - mean * mean
        scale = gam * lax.rsqrt(var + 1e-5)
        shift = bet - mean * scale
        sel = jnp.where((gam >= 0)[None, :], mx_ref[:, :cout],
                        mn_ref[:, :cout])
        h = (a_ref[...] + sel) * scale[None, :] + shift[None, :]
        out_ref[...] = jnp.where(h >= 0, h, 0.2 * h)


def _run_finalize(a2, s, mx, mn, s2, gamma, beta, bs=2048):
    npts, cout = a2.shape
    tw = s.shape[1]
    nw = s2.shape[0]
    count = float(npts * KNN)
    grid = (2, npts // bs)
    blk_a = pl.BlockSpec((bs, cout), lambda p, i: (i, 0))
    blk_w = pl.BlockSpec((bs, tw), lambda p, i: (i, 0))
    return pl.pallas_call(
        functools.partial(_finalize_body, count, cout),
        grid=grid,
        in_specs=[
            blk_a, blk_w, blk_w, blk_w,
            pl.BlockSpec((nw, tw), lambda p, i: (0, 0)),
            pl.BlockSpec((1, cout), lambda p, i: (0, 0)),
            pl.BlockSpec((1, cout), lambda p, i: (0, 0)),
        ],
        out_specs=blk_a,
        out_shape=jax.ShapeDtypeStruct((npts, cout), jnp.float32),
        scratch_shapes=[pltpu.VMEM((8, cout), jnp.float32)],
    )(a2, s, mx, mn, s2, gamma.reshape(1, -1), beta.reshape(1, -1))


def _sc_gather_reduce(bb_flat, idx2d):
    npts, tw = bb_flat.shape
    return _make_sc_gather_reduce(npts, tw // 2, 32)(bb_flat, idx2d)


def kernel(x, W, gamma, beta):
    b, c, n = x.shape
    cout = W.shape[0]
    xt = jnp.transpose(x, (0, 2, 1))
    w1 = W[:, :c]
    w2 = W[:, c:]
    wd = w1 - w2
    idx, a3, bb3 = _run_knn(xt, wd, w2)
    npts = b * n
    idx2d = jnp.transpose(idx[:, :KNN, :], (0, 2, 1)).reshape(
        (npts * KNN) // 128, 128)
    bb_flat = bb3.reshape(npts, 2 * cout)
    a2 = a3.reshape(npts, cout)
    s, mx, mn, s2 = _sc_gather_reduce(bb_flat, idx2d)
    out = _run_finalize(a2, s, mx, mn, s2, gamma, beta)
    return jnp.transpose(out.reshape(b, n, cout), (0, 2, 1))


# speedup vs baseline: 14.5957x; 1.2062x over previous
import functools

import jax
import jax.numpy as jnp
from jax import lax
from jax.experimental import pallas as pl
from jax.experimental.pallas import tpu as pltpu
from jax.experimental.pallas import tpu_sc as plsc

KNN = 20
KPAD = 32


def _knn_body(n_pts, rows, xa_ref, xr_ref, wd_ref, w2_ref,
              idx_ref, a_ref, bb_ref):
    b = pl.program_id(0)
    xa = xa_ref[0]
    xr = xr_ref[0]
    g = jnp.dot(xr, xa.T, preferred_element_type=jnp.float32)
    xx_a = jnp.sum(xa * xa, axis=1)
    xx_r = jnp.sum(xr * xr, axis=1)
    d = 2.0 * g - xx_r[:, None] - xx_a[None, :]
    base = b * n_pts
    colsf = jnp.where(
        lax.broadcasted_iota(jnp.int32, (n_pts, 8), 1) == 0,
        lax.broadcasted_iota(jnp.int32, (n_pts, 8), 0).astype(jnp.float32),
        0.0)
    m = jnp.max(d, axis=1, keepdims=True)
    for t in range(KNN):
        eq = d == m
        eqf = jnp.where(eq, 1.0, 0.0)
        idxv = jnp.dot(eqf, colsf, preferred_element_type=jnp.float32)
        amin = jnp.minimum(idxv[:, 0].astype(jnp.int32), n_pts - 1)
        idx_ref[0, t, :] = amin + base
        d = jnp.where(eq, -jnp.inf, d)
        if t < KNN - 1:
            m = jnp.max(d, axis=1, keepdims=True)
    a_ref[0] = jnp.dot(xr, wd_ref[...].T, preferred_element_type=jnp.float32)
    bb = jnp.dot(xr, w2_ref[...].T, preferred_element_type=jnp.float32)
    bb_ref[0] = jnp.concatenate([bb, jnp.zeros_like(bb)], axis=1)


def _run_knn(xt, wd, w2, rows=256):
    b, n, c = xt.shape
    cout = wd.shape[0]
    grid = (b, n // rows)
    return pl.pallas_call(
        functools.partial(_knn_body, n, rows),
        grid=grid,
        in_specs=[
            pl.BlockSpec((1, n, c), lambda bi, i: (bi, 0, 0)),
            pl.BlockSpec((1, rows, c), lambda bi, i: (bi, i, 0)),
            pl.BlockSpec((cout, c), lambda bi, i: (0, 0)),
            pl.BlockSpec((cout, c), lambda bi, i: (0, 0)),
        ],
        out_specs=[
            pl.BlockSpec((1, KPAD, rows), lambda bi, i: (bi, 0, i)),
            pl.BlockSpec((1, rows, cout), lambda bi, i: (bi, i, 0)),
            pl.BlockSpec((1, rows, 2 * cout), lambda bi, i: (bi, i, 0)),
        ],
        out_shape=[
            jax.ShapeDtypeStruct((b, KPAD, n), jnp.int32),
            jax.ShapeDtypeStruct((b, n, cout), jnp.float32),
            jax.ShapeDtypeStruct((b, n, 2 * cout), jnp.float32),
        ],
    )(xt, xt, wd, w2)


_P = 32
_GROUPS = 5
_LANE = 16


def _make_sc_gather_reduce(npts, cout, nworkers):
    per_w = npts // nworkers
    nsub = per_w // _P
    idx_rows_per_sub = (_P * KNN) // 128
    idx_rows_per_w = (per_w * KNN) // 128
    mesh = plsc.VectorSubcoreMesh(core_axis_name="c", subcore_axis_name="s")
    nc = plsc.get_sparse_core_info().num_cores
    ngrp = cout // _LANE
    tw = 2 * cout

    @functools.partial(
        pl.kernel, mesh=mesh,
        out_type=[
            jax.ShapeDtypeStruct((npts, tw), jnp.float32),
            jax.ShapeDtypeStruct((npts, tw), jnp.float32),
            jax.ShapeDtypeStruct((npts, tw), jnp.float32),
            jax.ShapeDtypeStruct((nworkers * 8, tw), jnp.float32),
        ],
        scratch_types=[
            pltpu.VMEM((idx_rows_per_w, 128), jnp.int32),
            pltpu.VMEM((_GROUPS, 128, tw), jnp.float32),
            pltpu.VMEM((_P, tw), jnp.float32),
            pltpu.VMEM((_P, tw), jnp.float32),
            pltpu.VMEM((_P, tw), jnp.float32),
            pltpu.VMEM((8, tw), jnp.float32),
            pltpu.SemaphoreType.DMA,
        ],
    )
    def sc_kernel(bb_hbm, idx_hbm, s_hbm, mx_hbm, mn_hbm, s2_hbm,
                  idx_v, rows_v, sv, mv, nv, s2v, sem):
        wid = lax.axis_index("s") * nc + lax.axis_index("c")
        pltpu.sync_copy(idx_hbm.at[pl.ds(wid * idx_rows_per_w,
                                         idx_rows_per_w)], idx_v)

        def sub_body(sub, s2c):
            base_pt = wid * per_w + sub * _P
            handles = [
                pltpu.async_copy(
                    bb_hbm.at[idx_v.at[sub * idx_rows_per_sub + g]],
                    rows_v.at[g], sem)
                for g in range(_GROUPS)
            ]
            for h in handles:
                h.wait()

            def p_body(p, s2i):
                s2i = list(s2i)
                s = [None] * ngrp
                mxa = [None] * ngrp
                mna = [None] * ngrp
                e0 = p * KNN
                for t in range(KNN):
                    e = e0 + t
                    g = e // 128
                    r = e - g * 128
                    for cgi in range(ngrp):
                        v = rows_v[g, r, pl.ds(cgi * _LANE, _LANE)]
                        if t == 0:
                            s[cgi] = v
                            mxa[cgi] = v
                            mna[cgi] = v
                        else:
                            s[cgi] = s[cgi] + v
                            mxa[cgi] = jnp.maximum(mxa[cgi], v)
                            mna[cgi] = jnp.minimum(mna[cgi], v)
                        s2i[cgi] = s2i[cgi] + v * v
                for cgi in range(ngrp):
                    sv[p, pl.ds(cgi * _LANE, _LANE)] = s[cgi]
                    mv[p, pl.ds(cgi * _LANE, _LANE)] = mxa[cgi]
                    nv[p, pl.ds(cgi * _LANE, _LANE)] = mna[cgi]
                return tuple(s2i)

            s2c = lax.fori_loop(0, _P, p_body, s2c)
            pltpu.sync_copy(sv, s_hbm.at[pl.ds(base_pt, _P)])
            pltpu.sync_copy(mv, mx_hbm.at[pl.ds(base_pt, _P)])
            pltpu.sync_copy(nv, mn_hbm.at[pl.ds(base_pt, _P)])
            return s2c

        zero = jnp.zeros((_LANE,), jnp.float32)
        s2c = lax.fori_loop(0, nsub, sub_body, (zero,) * ngrp)
        for r in range(8):
            for cgi in range(ngrp):
                s2v[r, pl.ds(cgi * _LANE, _LANE)] = (
                    s2c[cgi] if r == 0 else zero)
        pltpu.sync_copy(s2v, s2_hbm.at[pl.ds(wid * 8, 8)])

    return sc_kernel


def _finalize_body(count, cout, a_ref, s_ref, mx_ref, mn_ref, s2_ref,
                   gam_ref, bet_ref, out_ref, acc_ref):
    phase = pl.program_id(0)
    i = pl.program_id(1)

    @pl.when(jnp.logical_and(phase == 0, i == 0))
    def _():
        acc_ref[...] = jnp.zeros_like(acc_ref)

    @pl.when(phase == 0)
    def _():
        a = a_ref[...]
        s = s_ref[:, :cout]
        acc_ref[0, :] += jnp.sum(a, axis=0)
        acc_ref[1, :] += jnp.sum(a * a, axis=0)
        acc_ref[2, :] += jnp.sum(a * s, axis=0)
        acc_ref[3, :] += jnp.sum(s, axis=0)

    @pl.when(phase == 1)
    def _():
        gam = gam_ref[0]
        bet = bet_ref[0]
        s2sum = jnp.sum(s2_ref[:, :cout], axis=0)
        sumh = KNN * acc_ref[0, :] + acc_ref[3, :]
        sumh2 = KNN * acc_ref[1, :] + 2.0 * acc_ref[2, :] + s2sum
        mean = sumh / count
        var = sumh2 / count - mean * mean
        scale = gam * lax.rsqrt(var + 1e-5)
        shift = bet - mean * scale
        sel = jnp.where((gam >= 0)[None, :], mx_ref[:, :cout],
                        mn_ref[:, :cout])
        h = (a_ref[...] + sel) * scale[None, :] + shift[None, :]
        out_ref[...] = jnp.where(h >= 0, h, 0.2 * h)


def _run_finalize(a2, s, mx, mn, s2, gamma, beta, bs=2048):
    npts, cout = a2.shape
    tw = s.shape[1]
    nw = s2.shape[0]
    count = float(npts * KNN)
    grid = (2, npts // bs)
    blk_a = pl.BlockSpec((bs, cout), lambda p, i: (i, 0))
    blk_w = pl.BlockSpec((bs, tw), lambda p, i: (i, 0))
    return pl.pallas_call(
        functools.partial(_finalize_body, count, cout),
        grid=grid,
        in_specs=[
            blk_a, blk_w, blk_w, blk_w,
            pl.BlockSpec((nw, tw), lambda p, i: (0, 0)),
            pl.BlockSpec((1, cout), lambda p, i: (0, 0)),
            pl.BlockSpec((1, cout), lambda p, i: (0, 0)),
        ],
        out_specs=blk_a,
        out_shape=jax.ShapeDtypeStruct((npts, cout), jnp.float32),
        scratch_shapes=[pltpu.VMEM((8, cout), jnp.float32)],
    )(a2, s, mx, mn, s2, gamma.reshape(1, -1), beta.reshape(1, -1))


def _sc_gather_reduce(bb_flat, idx2d):
    npts, tw = bb_flat.shape
    return _make_sc_gather_reduce(npts, tw // 2, 32)(bb_flat, idx2d)


def kernel(x, W, gamma, beta):
    b, c, n = x.shape
    cout = W.shape[0]
    xt = jnp.transpose(x, (0, 2, 1))
    w1 = W[:, :c]
    w2 = W[:, c:]
    wd = w1 - w2
    idx, a3, bb3 = _run_knn(xt, wd, w2)
    npts = b * n
    idx2d = jnp.transpose(idx[:, :KNN, :], (0, 2, 1)).reshape(
        (npts * KNN) // 128, 128)
    bb_flat = bb3.reshape(npts, 2 * cout)
    a2 = a3.reshape(npts, cout)
    s, mx, mn, s2 = _sc_gather_reduce(bb_flat, idx2d)
    out = _run_finalize(a2, s, mx, mn, s2, gamma, beta)
    return jnp.transpose(out.reshape(b, n, cout), (0, 2, 1))


# speedup vs baseline: 14.9027x; 1.0210x over previous
import functools

import jax
import jax.numpy as jnp
from jax import lax
from jax.experimental import pallas as pl
from jax.experimental.pallas import tpu as pltpu
from jax.experimental.pallas import tpu_sc as plsc

KNN = 20
KPAD = 32


def _knn_body(n_pts, rows, xa_ref, xr_ref, wd_ref, w2_ref,
              idx_ref, a_ref, bb_ref):
    b = pl.program_id(0)
    xa = xa_ref[0]
    xr = xr_ref[0]
    g = jnp.dot(xr, xa.T, preferred_element_type=jnp.float32)
    xx_a = jnp.sum(xa * xa, axis=1)
    xx_r = jnp.sum(xr * xr, axis=1)
    d = 2.0 * g - xx_r[:, None] - xx_a[None, :]
    base = b * n_pts
    lane8 = lax.broadcasted_iota(jnp.int32, (n_pts, 8), 1)
    iota8 = lax.broadcasted_iota(jnp.int32, (n_pts, 8), 0)
    colsf = jnp.where(lane8 == 0, (iota8 // 64).astype(jnp.float32),
                      jnp.where(lane8 == 1, (iota8 % 64).astype(jnp.float32),
                                0.0))
    m = jnp.max(d, axis=1, keepdims=True)
    for t in range(KNN):
        eq = d == m
        eqf = jnp.where(eq, 1.0, 0.0)
        idxv = jnp.dot(eqf, colsf, preferred_element_type=jnp.float32)
        amin = (64.0 * idxv[:, 0] + idxv[:, 1] + 0.5).astype(jnp.int32)
        amin = jnp.minimum(amin, n_pts - 1)
        idx_ref[0, t, :] = amin + base
        d = jnp.where(eq, -jnp.inf, d)
        if t < KNN - 1:
            m = jnp.max(d, axis=1, keepdims=True)
    a_ref[0] = jnp.dot(xr, wd_ref[...].T, preferred_element_type=jnp.float32)
    bb = jnp.dot(xr, w2_ref[...].T, preferred_element_type=jnp.float32)
    bb_ref[0] = jnp.concatenate([bb, jnp.zeros_like(bb)], axis=1)


def _run_knn(xt, wd, w2, rows=256):
    b, n, c = xt.shape
    cout = wd.shape[0]
    grid = (b, n // rows)
    return pl.pallas_call(
        functools.partial(_knn_body, n, rows),
        grid=grid,
        in_specs=[
            pl.BlockSpec((1, n, c), lambda bi, i: (bi, 0, 0)),
            pl.BlockSpec((1, rows, c), lambda bi, i: (bi, i, 0)),
            pl.BlockSpec((cout, c), lambda bi, i: (0, 0)),
            pl.BlockSpec((cout, c), lambda bi, i: (0, 0)),
        ],
        out_specs=[
            pl.BlockSpec((1, KPAD, rows), lambda bi, i: (bi, 0, i)),
            pl.BlockSpec((1, rows, cout), lambda bi, i: (bi, i, 0)),
            pl.BlockSpec((1, rows, 2 * cout), lambda bi, i: (bi, i, 0)),
        ],
        out_shape=[
            jax.ShapeDtypeStruct((b, KPAD, n), jnp.int32),
            jax.ShapeDtypeStruct((b, n, cout), jnp.float32),
            jax.ShapeDtypeStruct((b, n, 2 * cout), jnp.float32),
        ],
    )(xt, xt, wd, w2)


_P = 32
_GROUPS = 5
_LANE = 16


def _make_sc_gather_reduce(npts, cout, nworkers):
    per_w = npts // nworkers
    nsub = per_w // _P
    idx_rows_per_sub = (_P * KNN) // 128
    idx_rows_per_w = (per_w * KNN) // 128
    mesh = plsc.VectorSubcoreMesh(core_axis_name="c", subcore_axis_name="s")
    nc = plsc.get_sparse_core_info().num_cores
    ngrp = cout // _LANE
    tw = 2 * cout

    @functools.partial(
        pl.kernel, mesh=mesh,
        out_type=[
            jax.ShapeDtypeStruct((npts, tw), jnp.float32),
            jax.ShapeDtypeStruct((npts, tw), jnp.float32),
            jax.ShapeDtypeStruct((npts, tw), jnp.float32),
            jax.ShapeDtypeStruct((nworkers * 8, tw), jnp.float32),
        ],
        scratch_types=[
            pltpu.VMEM((idx_rows_per_w, 128), jnp.int32),
            pltpu.VMEM((_GROUPS, 128, tw), jnp.float32),
            pltpu.VMEM((_P, tw), jnp.float32),
            pltpu.VMEM((_P, tw), jnp.float32),
            pltpu.VMEM((_P, tw), jnp.float32),
            pltpu.VMEM((8, tw), jnp.float32),
            pltpu.SemaphoreType.DMA,
        ],
    )
    def sc_kernel(bb_hbm, idx_hbm, s_hbm, mx_hbm, mn_hbm, s2_hbm,
                  idx_v, rows_v, sv, mv, nv, s2v, sem):
        wid = lax.axis_index("s") * nc + lax.axis_index("c")
        pltpu.sync_copy(idx_hbm.at[pl.ds(wid * idx_rows_per_w,
                                         idx_rows_per_w)], idx_v)

        def sub_body(sub, s2c):
            base_pt = wid * per_w + sub * _P
            handles = [
                pltpu.async_copy(
                    bb_hbm.at[idx_v.at[sub * idx_rows_per_sub + g]],
                    rows_v.at[g], sem)
                for g in range(_GROUPS)
            ]
            for h in handles:
                h.wait()

            def p_body(p, s2i):
                s2i = list(s2i)
                s = [None] * ngrp
                mxa = [None] * ngrp
                mna = [None] * ngrp
                e0 = p * KNN
                for t in range(KNN):
                    e = e0 + t
                    g = e // 128
                    r = e - g * 128
                    for cgi in range(ngrp):
                        v = rows_v[g, r, pl.ds(cgi * _LANE, _LANE)]
                        if t == 0:
                            s[cgi] = v
                            mxa[cgi] = v
                            mna[cgi] = v
                        else:
                            s[cgi] = s[cgi] + v
                            mxa[cgi] = jnp.maximum(mxa[cgi], v)
                            mna[cgi] = jnp.minimum(mna[cgi], v)
                        s2i[cgi] = s2i[cgi] + v * v
                for cgi in range(ngrp):
                    sv[p, pl.ds(cgi * _LANE, _LANE)] = s[cgi]
                    mv[p, pl.ds(cgi * _LANE, _LANE)] = mxa[cgi]
                    nv[p, pl.ds(cgi * _LANE, _LANE)] = mna[cgi]
                return tuple(s2i)

            s2c = lax.fori_loop(0, _P, p_body, s2c)
            pltpu.sync_copy(sv, s_hbm.at[pl.ds(base_pt, _P)])
            pltpu.sync_copy(mv, mx_hbm.at[pl.ds(base_pt, _P)])
            pltpu.sync_copy(nv, mn_hbm.at[pl.ds(base_pt, _P)])
            return s2c

        zero = jnp.zeros((_LANE,), jnp.float32)
        s2c = lax.fori_loop(0, nsub, sub_body, (zero,) * ngrp)
        for r in range(8):
            for cgi in range(ngrp):
                s2v[r, pl.ds(cgi * _LANE, _LANE)] = (
                    s2c[cgi] if r == 0 else zero)
        pltpu.sync_copy(s2v, s2_hbm.at[pl.ds(wid * 8, 8)])

    return sc_kernel


def _finalize_body(count, cout, a_ref, s_ref, mx_ref, mn_ref, s2_ref,
                   gam_ref, bet_ref, out_ref, acc_ref):
    phase = pl.program_id(0)
    i = pl.program_id(1)

    @pl.when(jnp.logical_and(phase == 0, i == 0))
    def _():
        acc_ref[...] = jnp.zeros_like(acc_ref)

    @pl.when(phase == 0)
    def _():
        a = a_ref[...]
        s = s_ref[:, :cout]
        acc_ref[0, :] += jnp.sum(a, axis=0)
        acc_ref[1, :] += jnp.sum(a * a, axis=0)
        acc_ref[2, :] += jnp.sum(a * s, axis=0)
        acc_ref[3, :] += jnp.sum(s, axis=0)

    @pl.when(phase == 1)
    def _():
        gam = gam_ref[0]
        bet = bet_ref[0]
        s2sum = jnp.sum(s2_ref[:, :cout], axis=0)
        sumh = KNN * acc_ref[0, :] + acc_ref[3, :]
        sumh2 = KNN * acc_ref[1, :] + 2.0 * acc_ref[2, :] + s2sum
        mean = sumh / count
        var = sumh2 / count - mean * mean
        scale = gam * lax.rsqrt(var + 1e-5)
        shift = bet - mean * scale
        sel = jnp.where((gam >= 0)[None, :], mx_ref[:, :cout],
                        mn_ref[:, :cout])
        h = (a_ref[...] + sel) * scale[None, :] + shift[None, :]
        out_ref[...] = jnp.where(h >= 0, h, 0.2 * h)


def _run_finalize(a2, s, mx, mn, s2, gamma, beta, bs=2048):
    npts, cout = a2.shape
    tw = s.shape[1]
    nw = s2.shape[0]
    count = float(npts * KNN)
    grid = (2, npts // bs)
    blk_a = pl.BlockSpec((bs, cout), lambda p, i: (i, 0))
    blk_w = pl.BlockSpec((bs, tw), lambda p, i: (i, 0))
    return pl.pallas_call(
        functools.partial(_finalize_body, count, cout),
        grid=grid,
        in_specs=[
            blk_a, blk_w, blk_w, blk_w,
            pl.BlockSpec((nw, tw), lambda p, i: (0, 0)),
            pl.BlockSpec((1, cout), lambda p, i: (0, 0)),
            pl.BlockSpec((1, cout), lambda p, i: (0, 0)),
        ],
        out_specs=blk_a,
        out_shape=jax.ShapeDtypeStruct((npts, cout), jnp.float32),
        scratch_shapes=[pltpu.VMEM((8, cout), jnp.float32)],
    )(a2, s, mx, mn, s2, gamma.reshape(1, -1), beta.reshape(1, -1))


def _sc_gather_reduce(bb_flat, idx2d):
    npts, tw = bb_flat.shape
    return _make_sc_gather_reduce(npts, tw // 2, 32)(bb_flat, idx2d)


def kernel(x, W, gamma, beta):
    b, c, n = x.shape
    cout = W.shape[0]
    xt = jnp.transpose(x, (0, 2, 1))
    w1 = W[:, :c]
    w2 = W[:, c:]
    wd = w1 - w2
    idx, a3, bb3 = _run_knn(xt, wd, w2)
    npts = b * n
    idx2d = jnp.transpose(idx[:, :KNN, :], (0, 2, 1)).reshape(
        (npts * KNN) // 128, 128)
    bb_flat = bb3.reshape(npts, 2 * cout)
    a2 = a3.reshape(npts, cout)
    s, mx, mn, s2 = _sc_gather_reduce(bb_flat, idx2d)
    out = _run_finalize(a2, s, mx, mn, s2, gamma, beta)
    return jnp.transpose(out.reshape(b, n, cout), (0, 2, 1))


# speedup vs baseline: 15.0453x; 1.0096x over previous
import functools

import jax
import jax.numpy as jnp
from jax import lax
from jax.experimental import pallas as pl
from jax.experimental.pallas import tpu as pltpu
from jax.experimental.pallas import tpu_sc as plsc

KNN = 20
KPAD = 32


def _knn_body(n_pts, rows, xa_ref, xr_ref, wd_ref, w2_ref,
              idx_ref, a_ref, bb_ref):
    b = pl.program_id(0)
    xa = xa_ref[0]
    xr = xr_ref[0]
    g = jnp.dot(xr, xa.T, preferred_element_type=jnp.float32)
    xx_a = jnp.sum(xa * xa, axis=1)
    xx_r = jnp.sum(xr * xr, axis=1)
    d = 2.0 * g - xx_r[:, None] - xx_a[None, :]
    base = b * n_pts
    lane8 = lax.broadcasted_iota(jnp.int32, (n_pts, 8), 1)
    iota8 = lax.broadcasted_iota(jnp.int32, (n_pts, 8), 0)
    colsf = jnp.where(lane8 == 0, (iota8 // 64).astype(jnp.float32),
                      jnp.where(lane8 == 1, (iota8 % 64).astype(jnp.float32),
                                0.0))
    m = jnp.max(d, axis=1, keepdims=True)
    for t in range(KNN):
        eq = d == m
        eqf = jnp.where(eq, 1.0, 0.0)
        idxv = jnp.dot(eqf, colsf, preferred_element_type=jnp.float32)
        amin = (64.0 * idxv[:, 0] + idxv[:, 1] + 0.5).astype(jnp.int32)
        idx_ref[0, t, :] = jnp.minimum(amin, n_pts - 1) + base
        d = jnp.where(eq, -jnp.inf, d)
        if t < KNN - 1:
            m = jnp.max(d, axis=1, keepdims=True)
    a_ref[0] = jnp.dot(xr, wd_ref[...].T, preferred_element_type=jnp.float32)
    bb_ref[0] = jnp.dot(xr, w2_ref[...].T, preferred_element_type=jnp.float32)


def _run_knn(xt, wd, w2, rows=256):
    b, n, c = xt.shape
    cout = wd.shape[0]
    grid = (b, n // rows)
    return pl.pallas_call(
        functools.partial(_knn_body, n, rows),
        grid=grid,
        in_specs=[
            pl.BlockSpec((1, n, c), lambda bi, i: (bi, 0, 0)),
            pl.BlockSpec((1, rows, c), lambda bi, i: (bi, i, 0)),
            pl.BlockSpec((cout, c), lambda bi, i: (0, 0)),
            pl.BlockSpec((cout, c), lambda bi, i: (0, 0)),
        ],
        out_specs=[
            pl.BlockSpec((1, KPAD, rows), lambda bi, i: (bi, 0, i)),
            pl.BlockSpec((1, rows, cout), lambda bi, i: (bi, i, 0)),
            pl.BlockSpec((1, rows, cout), lambda bi, i: (bi, i, 0)),
        ],
        out_shape=[
            jax.ShapeDtypeStruct((b, KPAD, n), jnp.int32),
            jax.ShapeDtypeStruct((b, n, cout), jnp.float32),
            jax.ShapeDtypeStruct((b, n, cout), jnp.float32),
        ],
    )(xt, xt, wd, w2)


_P = 32
_GROUPS = 5
_LANE = 16


def _make_sc_gather_reduce(npts, cout, nworkers):
    per_w = npts // nworkers
    nsub = per_w // _P
    idx_rows_per_sub = (_P * KNN) // 128
    idx_rows_per_w = (per_w * KNN) // 128
    mesh = plsc.VectorSubcoreMesh(core_axis_name="c", subcore_axis_name="s")
    nc = plsc.get_sparse_core_info().num_cores
    ngrp = cout // _LANE
    tw = cout

    @functools.partial(
        pl.kernel, mesh=mesh,
        out_type=[
            jax.ShapeDtypeStruct((npts, tw), jnp.float32),
            jax.ShapeDtypeStruct((npts, tw), jnp.float32),
            jax.ShapeDtypeStruct((npts, tw), jnp.float32),
            jax.ShapeDtypeStruct((nworkers * 8, tw), jnp.float32),
        ],
        scratch_types=[
            pltpu.VMEM((idx_rows_per_w, 128), jnp.int32),
            pltpu.VMEM((_GROUPS, 128, tw), jnp.float32),
            pltpu.VMEM((_P, tw), jnp.float32),
            pltpu.VMEM((_P, tw), jnp.float32),
            pltpu.VMEM((_P, tw), jnp.float32),
            pltpu.VMEM((8, tw), jnp.float32),
            pltpu.SemaphoreType.DMA,
        ],
        compiler_params=pltpu.CompilerParams(use_tc_tiling_on_sc=False),
    )
    def sc_kernel(bb_hbm, idx_hbm, s_hbm, mx_hbm, mn_hbm, s2_hbm,
                  idx_v, rows_v, sv, mv, nv, s2v, sem):
        wid = lax.axis_index("s") * nc + lax.axis_index("c")
        pltpu.sync_copy(idx_hbm.at[pl.ds(wid * idx_rows_per_w,
                                         idx_rows_per_w)], idx_v)

        def sub_body(sub, s2c):
            base_pt = wid * per_w + sub * _P
            handles = [
                pltpu.async_copy(
                    bb_hbm.at[idx_v.at[sub * idx_rows_per_sub + g]],
                    rows_v.at[g], sem)
                for g in range(_GROUPS)
            ]
            for h in handles:
                h.wait()

            def p_body(p, s2i):
                s2i = list(s2i)
                s = [None] * ngrp
                mxa = [None] * ngrp
                mna = [None] * ngrp
                e0 = p * KNN
                for t in range(KNN):
                    e = e0 + t
                    g = e // 128
                    r = e - g * 128
                    for cgi in range(ngrp):
                        v = rows_v[g, r, pl.ds(cgi * _LANE, _LANE)]
                        if t == 0:
                            s[cgi] = v
                            mxa[cgi] = v
                            mna[cgi] = v
                        else:
                            s[cgi] = s[cgi] + v
                            mxa[cgi] = jnp.maximum(mxa[cgi], v)
                            mna[cgi] = jnp.minimum(mna[cgi], v)
                        s2i[cgi] = s2i[cgi] + v * v
                for cgi in range(ngrp):
                    sv[p, pl.ds(cgi * _LANE, _LANE)] = s[cgi]
                    mv[p, pl.ds(cgi * _LANE, _LANE)] = mxa[cgi]
                    nv[p, pl.ds(cgi * _LANE, _LANE)] = mna[cgi]
                return tuple(s2i)

            s2c = lax.fori_loop(0, _P, p_body, s2c)
            pltpu.sync_copy(sv, s_hbm.at[pl.ds(base_pt, _P)])
            pltpu.sync_copy(mv, mx_hbm.at[pl.ds(base_pt, _P)])
            pltpu.sync_copy(nv, mn_hbm.at[pl.ds(base_pt, _P)])
            return s2c

        zero = jnp.zeros((_LANE,), jnp.float32)
        s2c = lax.fori_loop(0, nsub, sub_body, (zero,) * ngrp)
        for r in range(8):
            for cgi in range(ngrp):
                s2v[r, pl.ds(cgi * _LANE, _LANE)] = (
                    s2c[cgi] if r == 0 else zero)
        pltpu.sync_copy(s2v, s2_hbm.at[pl.ds(wid * 8, 8)])

    return sc_kernel


def _finalize_body(count, cout, a_ref, s_ref, mx_ref, mn_ref, s2_ref,
                   gam_ref, bet_ref, out_ref, acc_ref):
    phase = pl.program_id(0)
    i = pl.program_id(1)

    @pl.when(jnp.logical_and(phase == 0, i == 0))
    def _():
        acc_ref[...] = jnp.zeros_like(acc_ref)

    @pl.when(phase == 0)
    def _():
        a = a_ref[...]
        s = s_ref[:, :cout]
        acc_ref[0, :] += jnp.sum(a, axis=0)
        acc_ref[1, :] += jnp.sum(a * a, axis=0)
        acc_ref[2, :] += jnp.sum(a * s, axis=0)
        acc_ref[3, :] += jnp.sum(s, axis=0)

    @pl.when(phase == 1)
    def _():
        gam = gam_ref[0]
        bet = bet_ref[0]
        s2sum = jnp.sum(s2_ref[:, :cout], axis=0)
        sumh = KNN * acc_ref[0, :] + acc_ref[3, :]
        sumh2 = KNN * acc_ref[1, :] + 2.0 * acc_ref[2, :] + s2sum
        mean = sumh / count
        var = sumh2 / count - mean * mean
        scale = gam * lax.rsqrt(var + 1e-5)
        shift = bet - mean * scale
        sel = jnp.where((gam >= 0)[None, :], mx_ref[:, :cout],
                        mn_ref[:, :cout])
        h = (a_ref[...] + sel) * scale[None, :] + shift[None, :]
        out_ref[...] = jnp.where(h >= 0, h, 0.2 * h)


def _run_finalize(a2, s, mx, mn, s2, gamma, beta, bs=2048):
    npts, cout = a2.shape
    tw = s.shape[1]
    nw = s2.shape[0]
    count = float(npts * KNN)
    grid = (2, npts // bs)
    blk_a = pl.BlockSpec((bs, cout), lambda p, i: (i, 0))
    blk_w = pl.BlockSpec((bs, tw), lambda p, i: (i, 0))
    return pl.pallas_call(
        functools.partial(_finalize_body, count, cout),
        grid=grid,
        in_specs=[
            blk_a, blk_w, blk_w, blk_w,
            pl.BlockSpec((nw, tw), lambda p, i: (0, 0)),
            pl.BlockSpec((1, cout), lambda p, i: (0, 0)),
            pl.BlockSpec((1, cout), lambda p, i: (0, 0)),
        ],
        out_specs=blk_a,
        out_shape=jax.ShapeDtypeStruct((npts, cout), jnp.float32),
        scratch_shapes=[pltpu.VMEM((8, cout), jnp.float32)],
    )(a2, s, mx, mn, s2, gamma.reshape(1, -1), beta.reshape(1, -1))


def _sc_gather_reduce(bb_flat, idx2d):
    npts, cw = bb_flat.shape
    return _make_sc_gather_reduce(npts, cw, 32)(bb_flat, idx2d)


def kernel(x, W, gamma, beta):
    b, c, n = x.shape
    cout = W.shape[0]
    xt = jnp.transpose(x, (0, 2, 1))
    w1 = W[:, :c]
    w2 = W[:, c:]
    wd = w1 - w2
    idx, a3, bb3 = _run_knn(xt, wd, w2)
    npts = b * n
    idx2d = jnp.transpose(idx[:, :KNN, :], (0, 2, 1)).reshape(
        (npts * KNN) // 128, 128)
    bb_flat = bb3.reshape(npts, cout)
    a2 = a3.reshape(npts, cout)
    s, mx, mn, s2 = _sc_gather_reduce(bb_flat, idx2d)
    out = _run_finalize(a2, s, mx, mn, s2, gamma, beta)
    return jnp.transpose(out.reshape(b, n, cout), (0, 2, 1))


# speedup vs baseline: 15.4379x; 1.0261x over previous
import functools

import jax
import jax.numpy as jnp
from jax import lax
from jax.experimental import pallas as pl
from jax.experimental.pallas import tpu as pltpu
from jax.experimental.pallas import tpu_sc as plsc

KNN = 20
KPAD = 32


def _knn_body(n_pts, rows, xa_ref, xr_ref, wd_ref, w2_ref,
              idx_ref, a_ref, bb_ref):
    b = pl.program_id(0)
    xa = xa_ref[0]
    xr = xr_ref[0]
    g = jnp.dot(xr, xa.T, preferred_element_type=jnp.float32)
    xx_a = jnp.sum(xa * xa, axis=1)
    xx_r = jnp.sum(xr * xr, axis=1)
    d = 2.0 * g - xx_r[:, None] - xx_a[None, :]
    base = b * n_pts
    lane8 = lax.broadcasted_iota(jnp.int32, (n_pts, 8), 1)
    iota8 = lax.broadcasted_iota(jnp.int32, (n_pts, 8), 0)
    colsf = jnp.where(lane8 == 0, (iota8 // 64).astype(jnp.float32),
                      jnp.where(lane8 == 1, (iota8 % 64).astype(jnp.float32),
                                0.0))
    m = jnp.max(d, axis=1, keepdims=True)
    for t in range(KNN):
        eq = d == m
        eqf = jnp.where(eq, 1.0, 0.0)
        idxv = jnp.dot(eqf, colsf, preferred_element_type=jnp.float32)
        amin = (64.0 * idxv[:, 0] + idxv[:, 1] + 0.5).astype(jnp.int32)
        idx_ref[0, t, :] = jnp.minimum(amin, n_pts - 1) + base
        d = jnp.where(eq, -jnp.inf, d)
        if t < KNN - 1:
            m = jnp.max(d, axis=1, keepdims=True)
    a_ref[0] = jnp.dot(xr, wd_ref[...].T, preferred_element_type=jnp.float32)
    bb_ref[0] = jnp.dot(xr, w2_ref[...].T, preferred_element_type=jnp.float32)


def _run_knn(xt, wd, w2, rows=256):
    b, n, c = xt.shape
    cout = wd.shape[0]
    grid = (b, n // rows)
    return pl.pallas_call(
        functools.partial(_knn_body, n, rows),
        grid=grid,
        in_specs=[
            pl.BlockSpec((1, n, c), lambda bi, i: (bi, 0, 0)),
            pl.BlockSpec((1, rows, c), lambda bi, i: (bi, i, 0)),
            pl.BlockSpec((cout, c), lambda bi, i: (0, 0)),
            pl.BlockSpec((cout, c), lambda bi, i: (0, 0)),
        ],
        out_specs=[
            pl.BlockSpec((1, KPAD, rows), lambda bi, i: (bi, 0, i)),
            pl.BlockSpec((1, rows, cout), lambda bi, i: (bi, i, 0)),
            pl.BlockSpec((1, rows, cout), lambda bi, i: (bi, i, 0)),
        ],
        out_shape=[
            jax.ShapeDtypeStruct((b, KPAD, n), jnp.int32),
            jax.ShapeDtypeStruct((b, n, cout), jnp.float32),
            jax.ShapeDtypeStruct((b, n, cout), jnp.float32),
        ],
    )(xt, xt, wd, w2)


_P = 32
_GROUPS = 5
_LANE = 16


def _make_sc_gather_reduce(npts, cout, nworkers):
    per_w = npts // nworkers
    nsub = per_w // _P
    idx_rows_per_sub = (_P * KNN) // 128
    idx_rows_per_w = (per_w * KNN) // 128
    mesh = plsc.VectorSubcoreMesh(core_axis_name="c", subcore_axis_name="s")
    nc = plsc.get_sparse_core_info().num_cores
    ngrp = cout // _LANE
    tw = cout

    @functools.partial(
        pl.kernel, mesh=mesh,
        out_type=[
            jax.ShapeDtypeStruct((npts, tw), jnp.float32),
            jax.ShapeDtypeStruct((npts, tw), jnp.float32),
            jax.ShapeDtypeStruct((npts, tw), jnp.float32),
            jax.ShapeDtypeStruct((nworkers * 8, tw), jnp.float32),
        ],
        scratch_types=[
            pltpu.VMEM((idx_rows_per_w, 128), jnp.int32),
            pltpu.VMEM((_GROUPS, 128, tw), jnp.float32),
            pltpu.VMEM((_GROUPS, 128, tw), jnp.float32),
            pltpu.VMEM((_P, tw), jnp.float32),
            pltpu.VMEM((_P, tw), jnp.float32),
            pltpu.VMEM((_P, tw), jnp.float32),
            pltpu.VMEM((8, tw), jnp.float32),
            pltpu.SemaphoreType.DMA,
            pltpu.SemaphoreType.DMA,
        ],
        compiler_params=pltpu.CompilerParams(use_tc_tiling_on_sc=False),
    )
    def sc_kernel(bb_hbm, idx_hbm, s_hbm, mx_hbm, mn_hbm, s2_hbm,
                  idx_v, rows_v0, rows_v1, sv, mv, nv, s2v, sem0, sem1):
        wid = lax.axis_index("s") * nc + lax.axis_index("c")
        pltpu.sync_copy(idx_hbm.at[pl.ds(wid * idx_rows_per_w,
                                         idx_rows_per_w)], idx_v)

        def issue(sub, buf, sem):
            for g in range(_GROUPS):
                pltpu.async_copy(
                    bb_hbm.at[idx_v.at[sub * idx_rows_per_sub + g]],
                    buf.at[g], sem)

        def drain(sub, buf, sem):
            for g in range(_GROUPS):
                pltpu.make_async_copy(
                    bb_hbm.at[idx_v.at[sub * idx_rows_per_sub + g]],
                    buf.at[g], sem).wait()

        def compute(sub, buf, s2c):
            base_pt = wid * per_w + sub * _P

            def p_body(p, s2i):
                s2i = list(s2i)
                s = [None] * ngrp
                mxa = [None] * ngrp
                mna = [None] * ngrp
                e0 = p * KNN
                for t in range(KNN):
                    e = e0 + t
                    g = e // 128
                    r = e - g * 128
                    for cgi in range(ngrp):
                        v = buf[g, r, pl.ds(cgi * _LANE, _LANE)]
                        if t == 0:
                            s[cgi] = v
                            mxa[cgi] = v
                            mna[cgi] = v
                        else:
                            s[cgi] = s[cgi] + v
                            mxa[cgi] = jnp.maximum(mxa[cgi], v)
                            mna[cgi] = jnp.minimum(mna[cgi], v)
                        s2i[cgi] = s2i[cgi] + v * v
                for cgi in range(ngrp):
                    sv[p, pl.ds(cgi * _LANE, _LANE)] = s[cgi]
                    mv[p, pl.ds(cgi * _LANE, _LANE)] = mxa[cgi]
                    nv[p, pl.ds(cgi * _LANE, _LANE)] = mna[cgi]
                return tuple(s2i)

            s2c = lax.fori_loop(0, _P, p_body, s2c)
            pltpu.sync_copy(sv, s_hbm.at[pl.ds(base_pt, _P)])
            pltpu.sync_copy(mv, mx_hbm.at[pl.ds(base_pt, _P)])
            pltpu.sync_copy(nv, mn_hbm.at[pl.ds(base_pt, _P)])
            return s2c

        issue(0, rows_v0, sem0)

        def pair_body(p, s2c):
            sub0 = 2 * p
            issue(sub0 + 1, rows_v1, sem1)
            drain(sub0, rows_v0, sem0)
            s2c = compute(sub0, rows_v0, s2c)

            @pl.when(p < nsub // 2 - 1)
            def _():
                issue(sub0 + 2, rows_v0, sem0)

            drain(sub0 + 1, rows_v1, sem1)
            return compute(sub0 + 1, rows_v1, s2c)

        zero = jnp.zeros((_LANE,), jnp.float32)
        s2c = lax.fori_loop(0, nsub // 2, pair_body, (zero,) * ngrp)
        for r in range(8):
            for cgi in range(ngrp):
                s2v[r, pl.ds(cgi * _LANE, _LANE)] = (
                    s2c[cgi] if r == 0 else zero)
        pltpu.sync_copy(s2v, s2_hbm.at[pl.ds(wid * 8, 8)])

    return sc_kernel


def _finalize_body(count, cout, a_ref, s_ref, mx_ref, mn_ref, s2_ref,
                   gam_ref, bet_ref, out_ref, acc_ref):
    phase = pl.program_id(0)
    i = pl.program_id(1)

    @pl.when(jnp.logical_and(phase == 0, i == 0))
    def _():
        acc_ref[...] = jnp.zeros_like(acc_ref)

    @pl.when(phase == 0)
    def _():
        a = a_ref[...]
        s = s_ref[:, :cout]
        acc_ref[0, :] += jnp.sum(a, axis=0)
        acc_ref[1, :] += jnp.sum(a * a, axis=0)
        acc_ref[2, :] += jnp.sum(a * s, axis=0)
        acc_ref[3, :] += jnp.sum(s, axis=0)

    @pl.when(phase == 1)
    def _():
        gam = gam_ref[0]
        bet = bet_ref[0]
        s2sum = jnp.sum(s2_ref[:, :cout], axis=0)
        sumh = KNN * acc_ref[0, :] + acc_ref[3, :]
        sumh2 = KNN * acc_ref[1, :] + 2.0 * acc_ref[2, :] + s2sum
        mean = sumh / count
        var = sumh2 / count - mean * mean
        scale = gam * lax.rsqrt(var + 1e-5)
        shift = bet - mean * scale
        sel = jnp.where((gam >= 0)[None, :], mx_ref[:, :cout],
                        mn_ref[:, :cout])
        h = (a_ref[...] + sel) * scale[None, :] + shift[None, :]
        out_ref[...] = jnp.where(h >= 0, h, 0.2 * h)


def _run_finalize(a2, s, mx, mn, s2, gamma, beta, bs=2048):
    npts, cout = a2.shape
    tw = s.shape[1]
    nw = s2.shape[0]
    count = float(npts * KNN)
    grid = (2, npts // bs)
    blk_a = pl.BlockSpec((bs, cout), lambda p, i: (i, 0))
    blk_w = pl.BlockSpec((bs, tw), lambda p, i: (i, 0))
    return pl.pallas_call(
        functools.partial(_finalize_body, count, cout),
        grid=grid,
        in_specs=[
            blk_a, blk_w, blk_w, blk_w,
            pl.BlockSpec((nw, tw), lambda p, i: (0, 0)),
            pl.BlockSpec((1, cout), lambda p, i: (0, 0)),
            pl.BlockSpec((1, cout), lambda p, i: (0, 0)),
        ],
        out_specs=blk_a,
        out_shape=jax.ShapeDtypeStruct((npts, cout), jnp.float32),
        scratch_shapes=[pltpu.VMEM((8, cout), jnp.float32)],
    )(a2, s, mx, mn, s2, gamma.reshape(1, -1), beta.reshape(1, -1))


def _sc_gather_reduce(bb_flat, idx2d):
    npts, cw = bb_flat.shape
    return _make_sc_gather_reduce(npts, cw, 32)(bb_flat, idx2d)


def kernel(x, W, gamma, beta):
    b, c, n = x.shape
    cout = W.shape[0]
    xt = jnp.transpose(x, (0, 2, 1))
    w1 = W[:, :c]
    w2 = W[:, c:]
    wd = w1 - w2
    idx, a3, bb3 = _run_knn(xt, wd, w2)
    npts = b * n
    idx2d = jnp.transpose(idx[:, :KNN, :], (0, 2, 1)).reshape(
        (npts * KNN) // 128, 128)
    bb_flat = bb3.reshape(npts, cout)
    a2 = a3.reshape(npts, cout)
    s, mx, mn, s2 = _sc_gather_reduce(bb_flat, idx2d)
    out = _run_finalize(a2, s, mx, mn, s2, gamma, beta)
    return jnp.transpose(out.reshape(b, n, cout), (0, 2, 1))


# speedup vs baseline: 16.0209x; 1.0378x over previous
import functools

import jax
import jax.numpy as jnp
from jax import lax
from jax.experimental import pallas as pl
from jax.experimental.pallas import tpu as pltpu
from jax.experimental.pallas import tpu_sc as plsc

KNN = 20
KPAD = 32


def _knn_body(n_pts, rows, xa_ref, xr_ref, wd_ref, w2_ref,
              idx_ref, a_ref, bb_ref):
    b = pl.program_id(0)
    xa = xa_ref[0]
    xr = xr_ref[0]
    dn = (((0,), (0,)), ((), ()))
    g = lax.dot_general(xr, xa, dn,
                        preferred_element_type=jnp.float32)
    xx_a = jnp.sum(xa * xa, axis=0)
    xx_r = jnp.sum(xr * xr, axis=0)
    d = 2.0 * g - xx_r[:, None] - xx_a[None, :]
    base = b * n_pts
    lane8 = lax.broadcasted_iota(jnp.int32, (n_pts, 8), 1)
    iota8 = lax.broadcasted_iota(jnp.int32, (n_pts, 8), 0)
    colsf = jnp.where(lane8 == 0, (iota8 // 64).astype(jnp.float32),
                      jnp.where(lane8 == 1, (iota8 % 64).astype(jnp.float32),
                                0.0))
    m = jnp.max(d, axis=1, keepdims=True)
    for t in range(KNN):
        eq = d == m
        eqf = jnp.where(eq, 1.0, 0.0)
        idxv = jnp.dot(eqf, colsf, preferred_element_type=jnp.float32)
        amin = (64.0 * idxv[:, 0] + idxv[:, 1] + 0.5).astype(jnp.int32)
        idx_ref[0, t, :] = jnp.minimum(amin, n_pts - 1) + base
        d = jnp.where(eq, -jnp.inf, d)
        if t < KNN - 1:
            m = jnp.max(d, axis=1, keepdims=True)
    dnw = (((0,), (1,)), ((), ()))
    a_ref[0] = lax.dot_general(xr, wd_ref[...], dnw,
                               preferred_element_type=jnp.float32)
    bb_ref[0] = lax.dot_general(xr, w2_ref[...], dnw,
                                preferred_element_type=jnp.float32)


def _run_knn(x, wd, w2, rows=256):
    b, c, n = x.shape
    cout = wd.shape[0]
    grid = (b, n // rows)
    return pl.pallas_call(
        functools.partial(_knn_body, n, rows),
        grid=grid,
        in_specs=[
            pl.BlockSpec((1, c, n), lambda bi, i: (bi, 0, 0)),
            pl.BlockSpec((1, c, rows), lambda bi, i: (bi, 0, i)),
            pl.BlockSpec((cout, c), lambda bi, i: (0, 0)),
            pl.BlockSpec((cout, c), lambda bi, i: (0, 0)),
        ],
        out_specs=[
            pl.BlockSpec((1, KPAD, rows), lambda bi, i: (bi, 0, i)),
            pl.BlockSpec((1, rows, cout), lambda bi, i: (bi, i, 0)),
            pl.BlockSpec((1, rows, cout), lambda bi, i: (bi, i, 0)),
        ],
        out_shape=[
            jax.ShapeDtypeStruct((b, KPAD, n), jnp.int32),
            jax.ShapeDtypeStruct((b, n, cout), jnp.float32),
            jax.ShapeDtypeStruct((b, n, cout), jnp.float32),
        ],
    )(x, x, wd, w2)


_P = 32
_GROUPS = 5
_LANE = 16


def _make_sc_gather_reduce(npts, cout, nworkers):
    per_w = npts // nworkers
    nsub = per_w // _P
    idx_rows_per_sub = (_P * KNN) // 128
    idx_rows_per_w = (per_w * KNN) // 128
    mesh = plsc.VectorSubcoreMesh(core_axis_name="c", subcore_axis_name="s")
    nc = plsc.get_sparse_core_info().num_cores
    ngrp = cout // _LANE
    tw = cout

    @functools.partial(
        pl.kernel, mesh=mesh,
        out_type=[
            jax.ShapeDtypeStruct((npts, tw), jnp.float32),
            jax.ShapeDtypeStruct((npts, tw), jnp.float32),
            jax.ShapeDtypeStruct((npts, tw), jnp.float32),
            jax.ShapeDtypeStruct((nworkers * 8, tw), jnp.float32),
        ],
        scratch_types=[
            pltpu.VMEM((idx_rows_per_w, 128), jnp.int32),
            pltpu.VMEM((_GROUPS, 128, tw), jnp.float32),
            pltpu.VMEM((_GROUPS, 128, tw), jnp.float32),
            pltpu.VMEM((_P, tw), jnp.float32),
            pltpu.VMEM((_P, tw), jnp.float32),
            pltpu.VMEM((_P, tw), jnp.float32),
            pltpu.VMEM((8, tw), jnp.float32),
            pltpu.SemaphoreType.DMA,
            pltpu.SemaphoreType.DMA,
        ],
        compiler_params=pltpu.CompilerParams(use_tc_tiling_on_sc=False),
    )
    def sc_kernel(bb_hbm, idx_hbm, s_hbm, mx_hbm, mn_hbm, s2_hbm,
                  idx_v, rows_v0, rows_v1, sv, mv, nv, s2v, sem0, sem1):
        wid = lax.axis_index("s") * nc + lax.axis_index("c")
        pltpu.sync_copy(idx_hbm.at[pl.ds(wid * idx_rows_per_w,
                                         idx_rows_per_w)], idx_v)

        def issue(sub, buf, sem):
            for g in range(_GROUPS):
                pltpu.async_copy(
                    bb_hbm.at[idx_v.at[sub * idx_rows_per_sub + g]],
                    buf.at[g], sem)

        def drain(sub, buf, sem):
            for g in range(_GROUPS):
                pltpu.make_async_copy(
                    bb_hbm.at[idx_v.at[sub * idx_rows_per_sub + g]],
                    buf.at[g], sem).wait()

        def compute(sub, buf, s2c):
            base_pt = wid * per_w + sub * _P

            def p_body(p, s2i):
                s2i = list(s2i)
                s = [None] * ngrp
                mxa = [None] * ngrp
                mna = [None] * ngrp
                e0 = p * KNN
                for t in range(KNN):
                    e = e0 + t
                    g = e // 128
                    r = e - g * 128
                    for cgi in range(ngrp):
                        v = buf[g, r, pl.ds(cgi * _LANE, _LANE)]
                        if t == 0:
                            s[cgi] = v
                            mxa[cgi] = v
                            mna[cgi] = v
                        else:
                            s[cgi] = s[cgi] + v
                            mxa[cgi] = jnp.maximum(mxa[cgi], v)
                            mna[cgi] = jnp.minimum(mna[cgi], v)
                        s2i[cgi] = s2i[cgi] + v * v
                for cgi in range(ngrp):
                    sv[p, pl.ds(cgi * _LANE, _LANE)] = s[cgi]
                    mv[p, pl.ds(cgi * _LANE, _LANE)] = mxa[cgi]
                    nv[p, pl.ds(cgi * _LANE, _LANE)] = mna[cgi]
                return tuple(s2i)

            s2c = lax.fori_loop(0, _P, p_body, s2c)
            pltpu.sync_copy(sv, s_hbm.at[pl.ds(base_pt, _P)])
            pltpu.sync_copy(mv, mx_hbm.at[pl.ds(base_pt, _P)])
            pltpu.sync_copy(nv, mn_hbm.at[pl.ds(base_pt, _P)])
            return s2c

        issue(0, rows_v0, sem0)

        def pair_body(p, s2c):
            sub0 = 2 * p
            issue(sub0 + 1, rows_v1, sem1)
            drain(sub0, rows_v0, sem0)
            s2c = compute(sub0, rows_v0, s2c)

            @pl.when(p < nsub // 2 - 1)
            def _():
                issue(sub0 + 2, rows_v0, sem0)

            drain(sub0 + 1, rows_v1, sem1)
            return compute(sub0 + 1, rows_v1, s2c)

        zero = jnp.zeros((_LANE,), jnp.float32)
        s2c = lax.fori_loop(0, nsub // 2, pair_body, (zero,) * ngrp)
        for r in range(8):
            for cgi in range(ngrp):
                s2v[r, pl.ds(cgi * _LANE, _LANE)] = (
                    s2c[cgi] if r == 0 else zero)
        pltpu.sync_copy(s2v, s2_hbm.at[pl.ds(wid * 8, 8)])

    return sc_kernel


def _finalize_body(count, cout, a_ref, s_ref, mx_ref, mn_ref, s2_ref,
                   gam_ref, bet_ref, out_ref, acc_ref):
    phase = pl.program_id(0)
    i = pl.program_id(1)

    @pl.when(jnp.logical_and(phase == 0, i == 0))
    def _():
        acc_ref[...] = jnp.zeros_like(acc_ref)

    @pl.when(phase == 0)
    def _():
        a = a_ref[...]
        s = s_ref[:, :cout]
        acc_ref[0, :] += jnp.sum(a, axis=0)
        acc_ref[1, :] += jnp.sum(a * a, axis=0)
        acc_ref[2, :] += jnp.sum(a * s, axis=0)
        acc_ref[3, :] += jnp.sum(s, axis=0)

    @pl.when(phase == 1)
    def _():
        gam = gam_ref[0]
        bet = bet_ref[0]
        s2sum = jnp.sum(s2_ref[:, :cout], axis=0)
        sumh = KNN * acc_ref[0, :] + acc_ref[3, :]
        sumh2 = KNN * acc_ref[1, :] + 2.0 * acc_ref[2, :] + s2sum
        mean = sumh / count
        var = sumh2 / count - mean * mean
        scale = gam * lax.rsqrt(var + 1e-5)
        shift = bet - mean * scale
        sel = jnp.where((gam >= 0)[None, :], mx_ref[:, :cout],
                        mn_ref[:, :cout])
        h = (a_ref[...] + sel) * scale[None, :] + shift[None, :]
        h = jnp.where(h >= 0, h, 0.2 * h)
        out_ref[0] = h.T


def _run_finalize(a2, s, mx, mn, s2, gamma, beta, batches, bs=2048):
    npts, cout = a2.shape
    tw = s.shape[1]
    nw = s2.shape[0]
    n = npts // batches
    nb = n // bs
    count = float(npts * KNN)
    grid = (2, npts // bs)
    blk_a = pl.BlockSpec((bs, cout), lambda p, i: (i, 0))
    blk_w = pl.BlockSpec((bs, tw), lambda p, i: (i, 0))
    return pl.pallas_call(
        functools.partial(_finalize_body, count, cout),
        grid=grid,
        in_specs=[
            blk_a, blk_w, blk_w, blk_w,
            pl.BlockSpec((nw, tw), lambda p, i: (0, 0)),
            pl.BlockSpec((1, cout), lambda p, i: (0, 0)),
            pl.BlockSpec((1, cout), lambda p, i: (0, 0)),
        ],
        out_specs=pl.BlockSpec((1, cout, bs),
                               lambda p, i: (i // nb, 0, i % nb)),
        out_shape=jax.ShapeDtypeStruct((batches, cout, n), jnp.float32),
        scratch_shapes=[pltpu.VMEM((8, cout), jnp.float32)],
    )(a2, s, mx, mn, s2, gamma.reshape(1, -1), beta.reshape(1, -1))


def _sc_gather_reduce(bb_flat, idx2d):
    npts, cw = bb_flat.shape
    return _make_sc_gather_reduce(npts, cw, 32)(bb_flat, idx2d)


def kernel(x, W, gamma, beta):
    b, c, n = x.shape
    cout = W.shape[0]
    w1 = W[:, :c]
    w2 = W[:, c:]
    wd = w1 - w2
    idx, a3, bb3 = _run_knn(x, wd, w2)
    npts = b * n
    idx2d = jnp.transpose(idx[:, :KNN, :], (0, 2, 1)).reshape(
        (npts * KNN) // 128, 128)
    bb_flat = bb3.reshape(npts, cout)
    a2 = a3.reshape(npts, cout)
    s, mx, mn, s2 = _sc_gather_reduce(bb_flat, idx2d)
    return _run_finalize(a2, s, mx, mn, s2, gamma, beta, b)
